# baseline (speedup 1.0000x reference)
import jax
import jax.numpy as jnp
from jax.experimental import pallas as pl
from jax.experimental.pallas import tpu as pltpu
from jax.experimental.pallas import tpu_sc as plsc

B = 16384
D = 64
W = 128


def _sc_gather3(users, pos_items, neg_items, user_table, item_table):
    mesh = plsc.VectorSubcoreMesh(core_axis_name="core", subcore_axis_name="subcore")

    u2 = users.reshape(1, B)
    p2 = pos_items.reshape(1, B)
    n2 = neg_items.reshape(1, B)

    def run(u2, p2, n2, user_table, item_table):
        @pl.kernel(
            out_type=[
                jax.ShapeDtypeStruct((B, D), jnp.float32),
                jax.ShapeDtypeStruct((B, D), jnp.float32),
                jax.ShapeDtypeStruct((B, D), jnp.float32),
            ],
            mesh=mesh,
            compiler_params=pltpu.CompilerParams(use_tc_tiling_on_sc=False),
        )
        def k(u_hbm, p_hbm, n_hbm, ut_hbm, it_hbm, ou_hbm, op_hbm, on_hbm):
            def body(iu_v, ip_v, in_v, ou_v, op_v, on_v):
                pltpu.sync_copy(ut_hbm.at[iu_v.at[0]], ou_v)
                pltpu.sync_copy(it_hbm.at[ip_v.at[0]], op_v)
                pltpu.sync_copy(it_hbm.at[in_v.at[0]], on_v)

            pltpu.emit_pipeline(
                body,
                grid=(B // W,),
                in_specs=[
                    pl.BlockSpec((1, W), lambda i: (0, i)),
                    pl.BlockSpec((1, W), lambda i: (0, i)),
                    pl.BlockSpec((1, W), lambda i: (0, i)),
                ],
                out_specs=[
                    pl.BlockSpec((W, D), lambda i: (i, 0)),
                    pl.BlockSpec((W, D), lambda i: (i, 0)),
                    pl.BlockSpec((W, D), lambda i: (i, 0)),
                ],
                core_axis_name=("core", "subcore"),
                dimension_semantics=(pltpu.PARALLEL,),
            )(u_hbm, p_hbm, n_hbm, ou_hbm, op_hbm, on_hbm)

        return k(u2, p2, n2, user_table, item_table)

    return run(u2, p2, n2, user_table, item_table)


def _tc_norm(u, p, n):
    blk = 2048

    def body(u_ref, p_ref, n_ref, o_ref):
        uu = u_ref[...]
        pp = p_ref[...]
        nn = n_ref[...]
        o_ref[...] = (
            jnp.sum(uu * uu, axis=1, keepdims=True)
            + jnp.sum(pp * pp, axis=1, keepdims=True)
            + jnp.sum(nn * nn, axis=1, keepdims=True)
        )

    out = pl.pallas_call(
        body,
        grid=(B // blk,),
        in_specs=[
            pl.BlockSpec((blk, D), lambda i: (i, 0)),
            pl.BlockSpec((blk, D), lambda i: (i, 0)),
            pl.BlockSpec((blk, D), lambda i: (i, 0)),
        ],
        out_specs=pl.BlockSpec((blk, 1), lambda i: (i, 0)),
        out_shape=jax.ShapeDtypeStruct((B, 1), jnp.float32),
    )(u, p, n)
    return out.reshape(B)


def kernel(users, pos_items, neg_items, user_table, item_table):
    users = users.astype(jnp.int32)
    pos_items = pos_items.astype(jnp.int32)
    neg_items = neg_items.astype(jnp.int32)
    users_e, pos_e, neg_e = _sc_gather3(
        users, pos_items, neg_items, user_table, item_table
    )
    l2 = _tc_norm(users_e, pos_e, neg_e)
    return (users_e, pos_e, neg_e, l2)


# speedup vs baseline: 1.5501x; 1.5501x over previous
import jax
import jax.numpy as jnp
from jax import lax
from jax.experimental import pallas as pl
from jax.experimental.pallas import tpu as pltpu
from jax.experimental.pallas import tpu_sc as plsc

B = 16384
D = 64
N = 100000
CH = 2048
NC = 2
NW = 32


def _sc_gather3_t(users, pos_items, neg_items, ut_t, it_t):
    mesh = plsc.VectorSubcoreMesh(core_axis_name="core", subcore_axis_name="subcore")

    @pl.kernel(
        out_type=[
            jax.ShapeDtypeStruct((D, B), jnp.float32),
            jax.ShapeDtypeStruct((D, B), jnp.float32),
            jax.ShapeDtypeStruct((D, B), jnp.float32),
        ],
        mesh=mesh,
        compiler_params=pltpu.CompilerParams(needs_layout_passes=False),
        scratch_types=[
            pltpu.VMEM((N,), jnp.float32),
            pltpu.VMEM((CH,), jnp.int32),
            pltpu.VMEM((CH,), jnp.float32),
            pltpu.SemaphoreType.DMA,
        ],
    )
    def k(u_hbm, p_hbm, n_hbm, ut_hbm, it_hbm, ou_hbm, op_hbm, on_hbm,
          row_v, idx_v, out_v, sem):
        wid = lax.axis_index("subcore") * NC + lax.axis_index("core")

        def gather_pass(d, idx_hbm, out_hbm):
            for c in range(B // CH):
                pltpu.sync_copy(idx_hbm.at[pl.ds(c * CH, CH)], idx_v)

                @pl.loop(0, CH, step=16)
                def _(j):
                    iv = idx_v[pl.ds(j, 16)]
                    out_v[pl.ds(j, 16)] = plsc.load_gather(row_v, [iv])

                pltpu.sync_copy(out_v, out_hbm.at[d, pl.ds(c * CH, CH)])

        for kk in range(2):
            d = wid + NW * kk
            pltpu.async_copy(ut_hbm.at[d], row_v, sem).wait()
            gather_pass(d, u_hbm, ou_hbm)
            pltpu.async_copy(it_hbm.at[d], row_v, sem).wait()
            gather_pass(d, p_hbm, op_hbm)
            gather_pass(d, n_hbm, on_hbm)

    return k(users, pos_items, neg_items, ut_t, it_t)


def _tc_norm_t(u_t, p_t, n_t):
    blk = 2048

    def body(u_ref, p_ref, n_ref, o_ref):
        uu = u_ref[...]
        pp = p_ref[...]
        nn = n_ref[...]
        o_ref[...] = (
            jnp.sum(uu * uu, axis=0)
            + jnp.sum(pp * pp, axis=0)
            + jnp.sum(nn * nn, axis=0)
        )

    return pl.pallas_call(
        body,
        grid=(B // blk,),
        in_specs=[
            pl.BlockSpec((D, blk), lambda i: (0, i)),
            pl.BlockSpec((D, blk), lambda i: (0, i)),
            pl.BlockSpec((D, blk), lambda i: (0, i)),
        ],
        out_specs=pl.BlockSpec((blk,), lambda i: (i,)),
        out_shape=jax.ShapeDtypeStruct((B,), jnp.float32),
    )(u_t, p_t, n_t)


def kernel(users, pos_items, neg_items, user_table, item_table):
    users = users.astype(jnp.int32)
    pos_items = pos_items.astype(jnp.int32)
    neg_items = neg_items.astype(jnp.int32)
    ut_t = user_table.T
    it_t = item_table.T
    ou_t, op_t, on_t = _sc_gather3_t(users, pos_items, neg_items, ut_t, it_t)
    l2 = _tc_norm_t(ou_t, op_t, on_t)
    return (ou_t.T, op_t.T, on_t.T, l2)


# speedup vs baseline: 1.6869x; 1.0882x over previous
import jax
import jax.numpy as jnp
from jax import lax
from jax.experimental import pallas as pl
from jax.experimental.pallas import tpu as pltpu
from jax.experimental.pallas import tpu_sc as plsc

B = 16384
D = 64
N = 100000
CH = 8192
NC = 2
NW = 32


def _sc_gather3_t(users, pos_items, neg_items, ut_t, it_t):
    mesh = plsc.VectorSubcoreMesh(core_axis_name="core", subcore_axis_name="subcore")

    @pl.kernel(
        out_type=[
            jax.ShapeDtypeStruct((D, B), jnp.float32),
            jax.ShapeDtypeStruct((D, B), jnp.float32),
            jax.ShapeDtypeStruct((D, B), jnp.float32),
        ],
        mesh=mesh,
        compiler_params=pltpu.CompilerParams(needs_layout_passes=False),
        scratch_types=[
            pltpu.VMEM((N,), jnp.float32),
            pltpu.VMEM((CH,), jnp.int32),
            pltpu.VMEM((CH,), jnp.float32),
            pltpu.VMEM((CH,), jnp.float32),
            pltpu.SemaphoreType.DMA,
            pltpu.SemaphoreType.DMA,
        ],
    )
    def k(u_hbm, p_hbm, n_hbm, ut_hbm, it_hbm, ou_hbm, op_hbm, on_hbm,
          row_v, idx_v, out0_v, out1_v, sem_row, sem_out):
        wid = lax.axis_index("subcore") * NC + lax.axis_index("core")
        out_bufs = (out0_v, out1_v)

        def gather_pass(d, idx_hbm, out_hbm):
            handles = []
            for c in range(B // CH):
                pltpu.sync_copy(idx_hbm.at[pl.ds(c * CH, CH)], idx_v)
                ob = out_bufs[c % 2]

                @pl.loop(0, CH, step=16, unroll=8)
                def _(j):
                    iv = idx_v[pl.ds(j, 16)]
                    ob[pl.ds(j, 16)] = plsc.load_gather(row_v, [iv])

                handles.append(
                    pltpu.async_copy(ob, out_hbm.at[d, pl.ds(c * CH, CH)], sem_out)
                )
            for h in handles:
                h.wait()

        for kk in range(2):
            d = wid + NW * kk
            pltpu.async_copy(ut_hbm.at[d], row_v, sem_row).wait()
            gather_pass(d, u_hbm, ou_hbm)
            pltpu.async_copy(it_hbm.at[d], row_v, sem_row).wait()
            gather_pass(d, p_hbm, op_hbm)
            gather_pass(d, n_hbm, on_hbm)

    return k(users, pos_items, neg_items, ut_t, it_t)


def _tc_norm_t(u_t, p_t, n_t):
    blk = 2048

    def body(u_ref, p_ref, n_ref, o_ref):
        uu = u_ref[...]
        pp = p_ref[...]
        nn = n_ref[...]
        o_ref[...] = (
            jnp.sum(uu * uu, axis=0)
            + jnp.sum(pp * pp, axis=0)
            + jnp.sum(nn * nn, axis=0)
        )

    return pl.pallas_call(
        body,
        grid=(B // blk,),
        in_specs=[
            pl.BlockSpec((D, blk), lambda i: (0, i)),
            pl.BlockSpec((D, blk), lambda i: (0, i)),
            pl.BlockSpec((D, blk), lambda i: (0, i)),
        ],
        out_specs=pl.BlockSpec((blk,), lambda i: (i,)),
        out_shape=jax.ShapeDtypeStruct((B,), jnp.float32),
    )(u_t, p_t, n_t)


def kernel(users, pos_items, neg_items, user_table, item_table):
    users = users.astype(jnp.int32)
    pos_items = pos_items.astype(jnp.int32)
    neg_items = neg_items.astype(jnp.int32)
    ut_t = user_table.T
    it_t = item_table.T
    ou_t, op_t, on_t = _sc_gather3_t(users, pos_items, neg_items, ut_t, it_t)
    l2 = _tc_norm_t(ou_t, op_t, on_t)
    return (ou_t.T, op_t.T, on_t.T, l2)
